# baseline (speedup 1.0000x reference)
import functools

import jax
import jax.numpy as jnp
from jax import lax
from jax.experimental import pallas as pl
from jax.experimental.pallas import tpu as pltpu
from jax.experimental.pallas import tpu_sc as plsc

_T, _H, _I, _E = 2048, 2048, 1024, 8
_BT = 256
_BTR = 256
_NTR = 24
_TPR = _NTR * _BTR
_LN = 128
_NC, _NS = 2, 16
_NW = _NC * _NS
_TPW = _T // _NW
_CH = 16


def _router_body(x_ref, rw_ref, bias_ref, r0_ref, r1_ref, w0_ref, w1_ref,
                 meta_ref, xb_ref):
    x = x_ref[...]
    xb_ref[...] = x.astype(jnp.bfloat16)
    logits = jnp.dot(x, rw_ref[...], preferred_element_type=jnp.float32)
    col = lax.broadcasted_iota(jnp.int32, (_T, _LN), 1)
    g = jax.nn.sigmoid(logits + bias_ref[...])
    g = jnp.where(col < _E, g, -1.0)
    top1 = jnp.max(g, axis=1, keepdims=True)
    i1 = jnp.min(jnp.where(g == top1, col, _LN), axis=1, keepdims=True)
    oh0 = col == i1
    g2 = jnp.where(oh0, -1.0, g)
    top2 = jnp.max(g2, axis=1, keepdims=True)
    i2 = jnp.min(jnp.where(g2 == top2, col, _LN), axis=1, keepdims=True)
    oh1 = col == i2
    den = top1 + top2 + 1e-9
    w0_ref[...] = jnp.broadcast_to(top1 / den, (_T, _LN))
    w1_ref[...] = jnp.broadcast_to(top2 / den, (_T, _LN))
    cnt = oh0.astype(jnp.float32) + oh1.astype(jnp.float32)
    counts = jnp.sum(cnt, axis=0, keepdims=True)
    padded = jnp.floor((counts + (_BTR - 1)) * (1.0 / _BTR)) * _BTR
    ea = lax.broadcasted_iota(jnp.int32, (_LN, _LN), 0)
    eb = lax.broadcasted_iota(jnp.int32, (_LN, _LN), 1)
    off = jnp.dot(padded, (ea < eb).astype(jnp.float32),
                  preferred_element_type=jnp.float32)
    ta = lax.broadcasted_iota(jnp.int32, (_T, _T), 0)
    tb = lax.broadcasted_iota(jnp.int32, (_T, _T), 1)
    cumex = jnp.dot((tb < ta).astype(jnp.float32), cnt,
                    preferred_element_type=jnp.float32)
    pos = off + cumex
    r0 = jnp.sum(jnp.where(oh0, pos, 0.0), axis=1, keepdims=True)
    r1 = jnp.sum(jnp.where(oh1, pos, 0.0), axis=1, keepdims=True)
    r0_ref[...] = r0.astype(jnp.int32)
    r1_ref[...] = r1.astype(jnp.int32)
    total = jnp.sum(padded)
    lane = lax.broadcasted_iota(jnp.int32, (1, _LN), 1)
    start = lane.astype(jnp.float32) * _BTR
    eid = jnp.zeros((1, _LN), jnp.int32)
    for e in range(_E):
        eid = eid + (start >= off[0, e] + padded[0, e]).astype(jnp.int32)
    eid = jnp.minimum(eid, _E - 1)
    valid = ((lane - 64).astype(jnp.float32) * _BTR < total).astype(jnp.int32)
    meta_ref[...] = jnp.where(lane < 64, eid, valid)


_router_call = pl.pallas_call(
    _router_body,
    out_shape=(
        jax.ShapeDtypeStruct((_T, 1), jnp.int32),
        jax.ShapeDtypeStruct((_T, 1), jnp.int32),
        jax.ShapeDtypeStruct((_T, _LN), jnp.float32),
        jax.ShapeDtypeStruct((_T, _LN), jnp.float32),
        jax.ShapeDtypeStruct((1, _LN), jnp.int32),
        jax.ShapeDtypeStruct((_T, _H), jnp.bfloat16),
    ),
)


def _shared_body(x_ref, gw_ref, uw_ref, dw_ref, g0_ref, g1_ref, y_ref):
    x = x_ref[...]
    g = jnp.dot(x, gw_ref[0], preferred_element_type=jnp.float32)
    u = jnp.dot(x, uw_ref[0], preferred_element_type=jnp.float32)
    a = (g * jax.nn.sigmoid(g)) * u
    y = jnp.dot(a, dw_ref[0], preferred_element_type=jnp.float32)
    y_ref[...] = y + g0_ref[...].astype(jnp.float32) \
        + g1_ref[...].astype(jnp.float32)


_shared_call = pl.pallas_call(
    _shared_body,
    grid=(_T // _BT,),
    in_specs=[
        pl.BlockSpec((_BT, _H), lambda n: (n, 0)),
        pl.BlockSpec((1, _H, _I), lambda n: (0, 0, 0)),
        pl.BlockSpec((1, _H, _I), lambda n: (0, 0, 0)),
        pl.BlockSpec((1, _I, _H), lambda n: (0, 0, 0)),
        pl.BlockSpec((_BT, _H), lambda n: (n, 0)),
        pl.BlockSpec((_BT, _H), lambda n: (n, 0)),
    ],
    out_specs=pl.BlockSpec((_BT, _H), lambda n: (n, 0)),
    out_shape=jax.ShapeDtypeStruct((_T, _H), jnp.float32),
)


def _routed_body(meta_ref, x_ref, gw_ref, uw_ref, dw_ref, w_ref, y_ref):
    n = pl.program_id(0)

    @pl.when(meta_ref[64 + n] == 1)
    def _():
        x = x_ref[...].astype(jnp.float32)
        g = jnp.dot(x, gw_ref[0], preferred_element_type=jnp.float32)
        u = jnp.dot(x, uw_ref[0], preferred_element_type=jnp.float32)
        a = (g * jax.nn.sigmoid(g)) * u
        y = jnp.dot(a, dw_ref[0], preferred_element_type=jnp.float32)
        y_ref[...] = (y * w_ref[:, 0:1]).astype(jnp.bfloat16)


_routed_call = pl.pallas_call(
    _routed_body,
    grid_spec=pltpu.PrefetchScalarGridSpec(
        num_scalar_prefetch=1,
        grid=(_NTR,),
        in_specs=[
            pl.BlockSpec((_BTR, _H), lambda n, meta: (n, 0)),
            pl.BlockSpec((1, _H, _I), lambda n, meta: (meta[n], 0, 0)),
            pl.BlockSpec((1, _H, _I), lambda n, meta: (meta[n], 0, 0)),
            pl.BlockSpec((1, _I, _H), lambda n, meta: (meta[n], 0, 0)),
            pl.BlockSpec((_BTR, _LN), lambda n, meta: (n, 0)),
        ],
        out_specs=pl.BlockSpec((_BTR, _H), lambda n, meta: (n, 0)),
    ),
    out_shape=jax.ShapeDtypeStruct((_TPR, _H), jnp.bfloat16),
)

@functools.lru_cache(maxsize=None)
def _sc_kernels():
    mesh = plsc.VectorSubcoreMesh(core_axis_name="c", subcore_axis_name="s",
                                  num_cores=_NC, num_subcores=_NS)

    n_chunks = _TPW // _CH

    @functools.partial(
        pl.kernel,
        out_type=(
            jax.ShapeDtypeStruct((_TPR, _H // 2), jnp.int32),
            jax.ShapeDtypeStruct((_TPR, _LN), jnp.float32),
        ),
        mesh=mesh,
        scratch_types=[
            pltpu.VMEM((_TPW,), jnp.int32),
            pltpu.VMEM((_TPW,), jnp.int32),
            pltpu.VMEM((_TPW, _LN), jnp.float32),
            pltpu.VMEM((_TPW, _LN), jnp.float32),
            pltpu.VMEM((_CH, _H // 2), jnp.int32),
            pltpu.VMEM((_CH, _H // 2), jnp.int32),
            pltpu.SemaphoreType.DMA,
            pltpu.SemaphoreType.DMA,
            pltpu.SemaphoreType.DMA,
            pltpu.SemaphoreType.DMA,
            pltpu.SemaphoreType.DMA,
        ],
    )
    def sc_dispatch(xf_hbm, r0_hbm, r1_hbm, w0_hbm, w1_hbm, xs_hbm, ws_hbm,
                    r0_all, r1_all, wv0_all, wv1_all, xb0, xb1,
                    sem_in, semx0, semx1, sems0, sems1):
        wid = lax.axis_index("s") * _NC + lax.axis_index("c")
        base = wid * _TPW
        xbufs = (xb0, xb1)
        semx = (semx0, semx1)
        semsc = (sems0, sems1)
        pre = [
            pltpu.async_copy(r0_hbm.at[pl.ds(base, _TPW)], r0_all, sem_in),
            pltpu.async_copy(r1_hbm.at[pl.ds(base, _TPW)], r1_all, sem_in),
            pltpu.async_copy(w0_hbm.at[pl.ds(base, _TPW)], wv0_all, sem_in),
            pltpu.async_copy(w1_hbm.at[pl.ds(base, _TPW)], wv1_all, sem_in),
        ]
        xl = {0: pltpu.async_copy(xf_hbm.at[pl.ds(base, _CH)], xbufs[0],
                                  semx[0])}
        for cp in pre:
            cp.wait()
        sc_pend = {}
        for ci in range(n_chunks):
            b = ci % 2
            if ci >= 1:
                for cp in sc_pend.pop(ci - 1):
                    cp.wait()
            if ci + 1 < n_chunks:
                t1 = base + (ci + 1) * _CH
                xl[ci + 1] = pltpu.async_copy(
                    xf_hbm.at[pl.ds(t1, _CH)], xbufs[(ci + 1) % 2],
                    semx[(ci + 1) % 2])
            xl.pop(ci).wait()
            i0 = r0_all[pl.ds(ci * _CH, _CH)]
            i1 = r1_all[pl.ds(ci * _CH, _CH)]
            sc_pend[ci] = [
                pltpu.async_copy(xbufs[b], xs_hbm.at[i0], semsc[b]),
                pltpu.async_copy(xbufs[b], xs_hbm.at[i1], semsc[b]),
                pltpu.async_copy(wv0_all.at[pl.ds(ci * _CH, _CH)],
                                 ws_hbm.at[i0], semsc[b]),
                pltpu.async_copy(wv1_all.at[pl.ds(ci * _CH, _CH)],
                                 ws_hbm.at[i1], semsc[b]),
            ]
        for cp in sc_pend.pop(n_chunks - 1):
            cp.wait()

    n_jobs = 2 * (_TPW // _CH)

    @functools.partial(
        pl.kernel,
        out_type=(
            jax.ShapeDtypeStruct((_T, _H // 2), jnp.int32),
            jax.ShapeDtypeStruct((_T, _H // 2), jnp.int32),
        ),
        mesh=mesh,
        scratch_types=[
            pltpu.VMEM((_TPW,), jnp.int32),
            pltpu.VMEM((_TPW,), jnp.int32),
            pltpu.VMEM((_CH, _H // 2), jnp.int32),
            pltpu.VMEM((_CH, _H // 2), jnp.int32),
            pltpu.SemaphoreType.DMA,
            pltpu.SemaphoreType.DMA,
            pltpu.SemaphoreType.DMA,
            pltpu.SemaphoreType.DMA,
            pltpu.SemaphoreType.DMA,
        ],
    )
    def sc_combine(yr_hbm, r0_hbm, r1_hbm, g0_hbm, g1_hbm,
                   r0_all, r1_all, buf0, buf1, sem_in,
                   semg0, semg1, semo0, semo1):
        wid = lax.axis_index("s") * _NC + lax.axis_index("c")
        base = wid * _TPW
        bufs = (buf0, buf1)
        semg = (semg0, semg1)
        semo = (semo0, semo1)
        pre = [
            pltpu.async_copy(r0_hbm.at[pl.ds(base, _TPW)], r0_all, sem_in),
            pltpu.async_copy(r1_hbm.at[pl.ds(base, _TPW)], r1_all, sem_in),
        ]
        for cp in pre:
            cp.wait()

        def job_src(j):
            ci, k = divmod(j, 2)
            idx_ref = r0_all if k == 0 else r1_all
            dst = g0_hbm if k == 0 else g1_hbm
            return ci, idx_ref, dst

        def issue_gather(j):
            ci, idx_ref, _ = job_src(j)
            iv = idx_ref[pl.ds(ci * _CH, _CH)]
            return pltpu.async_copy(yr_hbm.at[iv], bufs[j % 2], semg[j % 2])

        g_pend = {0: issue_gather(0)}
        o_pend = {}
        for j in range(n_jobs):
            b = j % 2
            if j + 1 < n_jobs:
                if j >= 1:
                    o_pend.pop(j - 1).wait()
                g_pend[j + 1] = issue_gather(j + 1)
            g_pend.pop(j).wait()
            ci, _, dst = job_src(j)
            o_pend[j] = pltpu.async_copy(
                bufs[b], dst.at[pl.ds(base + ci * _CH, _CH)], semo[b])
        o_pend.pop(n_jobs - 1).wait()

    return sc_dispatch, sc_combine


def _as_i32(a):
    n, m = a.shape
    return lax.bitcast_convert_type(a.reshape(n, m // 2, 2), jnp.int32)


def _as_bf16(a):
    n, m = a.shape
    return lax.bitcast_convert_type(a, jnp.bfloat16).reshape(n, 2 * m)


def kernel(x, shared_gate, shared_up, shared_down, routed_gate, routed_up,
           routed_down, router_w, expert_bias):
    b, s, h = x.shape
    xf = x.reshape(-1, h)
    rw = jnp.pad(router_w, ((0, 0), (0, _LN - _E)))
    bias = jnp.pad(expert_bias, (0, _LN - _E)).reshape(1, _LN)
    r0, r1, w0, w1, meta, xb = _router_call(xf, rw, bias)
    r0f = r0.reshape(_T)
    r1f = r1.reshape(_T)
    meta_flat = meta.reshape(_LN)
    sc_dispatch, sc_combine = _sc_kernels()
    xs32, ws = sc_dispatch(_as_i32(xb), r0f, r1f, w0, w1)
    yr = _routed_call(meta_flat, _as_bf16(xs32), routed_gate, routed_up,
                      routed_down, ws)
    g0_32, g1_32 = sc_combine(_as_i32(yr), r0f, r1f)
    out = _shared_call(xf, shared_gate, shared_up, shared_down,
                       _as_bf16(g0_32), _as_bf16(g1_32))
    aux_loss = jnp.asarray(0.0, dtype=x.dtype)
    return (out.reshape(b, s, h), aux_loss)


# speedup vs baseline: 4.2914x; 4.2914x over previous
import functools

import jax
import jax.numpy as jnp
from jax import lax
from jax.experimental import pallas as pl
from jax.experimental.pallas import tpu as pltpu
from jax.experimental.pallas import tpu_sc as plsc

_T, _H, _I, _E = 2048, 2048, 1024, 8
_BT = 256
_BTR = 256
_NTR = 24
_TPR = _NTR * _BTR
_LN = 128
_NC, _NS = 2, 16
_NW = _NC * _NS
_TPW = _T // _NW
_CH = 16


def _pack_bf16(a):
    n = a.shape[1] // 2
    lo = lax.bitcast_convert_type(a[:, :n].astype(jnp.bfloat16), jnp.uint16)
    hi = lax.bitcast_convert_type(a[:, n:].astype(jnp.bfloat16), jnp.uint16)
    w = lo.astype(jnp.uint32) | (hi.astype(jnp.uint32) << 16)
    return lax.bitcast_convert_type(w, jnp.int32)


def _unpack_bf16(w):
    wu = lax.bitcast_convert_type(w, jnp.uint32)
    lo = lax.bitcast_convert_type((wu & 0xFFFF).astype(jnp.uint16),
                                  jnp.bfloat16)
    hi = lax.bitcast_convert_type((wu >> 16).astype(jnp.uint16), jnp.bfloat16)
    return jnp.concatenate([lo.astype(jnp.float32), hi.astype(jnp.float32)],
                           axis=1)


def _router_body(x_ref, rw_ref, bias_ref, r0_ref, r1_ref, w0_ref, w1_ref,
                 meta_ref, xb_ref):
    x = x_ref[...]
    xb_ref[...] = _pack_bf16(x)
    logits = jnp.dot(x, rw_ref[...], preferred_element_type=jnp.float32)
    col = lax.broadcasted_iota(jnp.int32, (_T, _LN), 1)
    g = jax.nn.sigmoid(logits + bias_ref[...])
    g = jnp.where(col < _E, g, -1.0)
    top1 = jnp.max(g, axis=1, keepdims=True)
    i1 = jnp.min(jnp.where(g == top1, col, _LN), axis=1, keepdims=True)
    oh0 = col == i1
    g2 = jnp.where(oh0, -1.0, g)
    top2 = jnp.max(g2, axis=1, keepdims=True)
    i2 = jnp.min(jnp.where(g2 == top2, col, _LN), axis=1, keepdims=True)
    oh1 = col == i2
    den = top1 + top2 + 1e-9
    w0_ref[...] = jnp.broadcast_to(top1 / den, (_T, _LN))
    w1_ref[...] = jnp.broadcast_to(top2 / den, (_T, _LN))
    cnt = oh0.astype(jnp.float32) + oh1.astype(jnp.float32)
    counts = jnp.sum(cnt, axis=0, keepdims=True)
    padded = jnp.floor((counts + (_BTR - 1)) * (1.0 / _BTR)) * _BTR
    ea = lax.broadcasted_iota(jnp.int32, (_LN, _LN), 0)
    eb = lax.broadcasted_iota(jnp.int32, (_LN, _LN), 1)
    off = jnp.dot(padded, (ea < eb).astype(jnp.float32),
                  preferred_element_type=jnp.float32)
    ta = lax.broadcasted_iota(jnp.int32, (_T, _T), 0)
    tb = lax.broadcasted_iota(jnp.int32, (_T, _T), 1)
    cumex = jnp.dot((tb < ta).astype(jnp.float32), cnt,
                    preferred_element_type=jnp.float32)
    pos = off + cumex
    r0 = jnp.sum(jnp.where(oh0, pos, 0.0), axis=1, keepdims=True)
    r1 = jnp.sum(jnp.where(oh1, pos, 0.0), axis=1, keepdims=True)
    r0_ref[...] = r0.astype(jnp.int32)
    r1_ref[...] = r1.astype(jnp.int32)
    total = jnp.sum(padded)
    lane = lax.broadcasted_iota(jnp.int32, (1, _LN), 1)
    start = lane.astype(jnp.float32) * _BTR
    eid = jnp.zeros((1, _LN), jnp.int32)
    for e in range(_E):
        eid = eid + (start >= off[0, e] + padded[0, e]).astype(jnp.int32)
    eid = jnp.minimum(eid, _E - 1)
    valid = ((lane - 64).astype(jnp.float32) * _BTR < total).astype(jnp.int32)
    meta_ref[...] = jnp.where(lane < 64, eid, valid)


_router_call = pl.pallas_call(
    _router_body,
    out_shape=(
        jax.ShapeDtypeStruct((_T, 1), jnp.int32),
        jax.ShapeDtypeStruct((_T, 1), jnp.int32),
        jax.ShapeDtypeStruct((_T, _LN), jnp.float32),
        jax.ShapeDtypeStruct((_T, _LN), jnp.float32),
        jax.ShapeDtypeStruct((1, _LN), jnp.int32),
        jax.ShapeDtypeStruct((_T, _H // 2), jnp.int32),
    ),
)


def _shared_body(x_ref, gw_ref, uw_ref, dw_ref, g0_ref, g1_ref, y_ref):
    x = x_ref[...]
    g = jnp.dot(x, gw_ref[0], preferred_element_type=jnp.float32)
    u = jnp.dot(x, uw_ref[0], preferred_element_type=jnp.float32)
    a = (g * jax.nn.sigmoid(g)) * u
    y = jnp.dot(a, dw_ref[0], preferred_element_type=jnp.float32)
    y_ref[...] = y + _unpack_bf16(g0_ref[...]) + _unpack_bf16(g1_ref[...])


_shared_call = pl.pallas_call(
    _shared_body,
    grid=(_T // _BT,),
    in_specs=[
        pl.BlockSpec((_BT, _H), lambda n: (n, 0)),
        pl.BlockSpec((1, _H, _I), lambda n: (0, 0, 0)),
        pl.BlockSpec((1, _H, _I), lambda n: (0, 0, 0)),
        pl.BlockSpec((1, _I, _H), lambda n: (0, 0, 0)),
        pl.BlockSpec((_BT, _H // 2), lambda n: (n, 0)),
        pl.BlockSpec((_BT, _H // 2), lambda n: (n, 0)),
    ],
    out_specs=pl.BlockSpec((_BT, _H), lambda n: (n, 0)),
    out_shape=jax.ShapeDtypeStruct((_T, _H), jnp.float32),
)


def _routed_body(meta_ref, x_ref, gw_ref, uw_ref, dw_ref, w_ref, y_ref):
    n = pl.program_id(0)

    @pl.when(meta_ref[64 + n] == 1)
    def _():
        x = _unpack_bf16(x_ref[...])
        g = jnp.dot(x, gw_ref[0], preferred_element_type=jnp.float32)
        u = jnp.dot(x, uw_ref[0], preferred_element_type=jnp.float32)
        a = (g * jax.nn.sigmoid(g)) * u
        y = jnp.dot(a, dw_ref[0], preferred_element_type=jnp.float32)
        y_ref[...] = _pack_bf16(y * w_ref[:, 0:1])


_routed_call = pl.pallas_call(
    _routed_body,
    grid_spec=pltpu.PrefetchScalarGridSpec(
        num_scalar_prefetch=1,
        grid=(_NTR,),
        in_specs=[
            pl.BlockSpec((_BTR, _H // 2), lambda n, meta: (n, 0)),
            pl.BlockSpec((1, _H, _I), lambda n, meta: (meta[n], 0, 0)),
            pl.BlockSpec((1, _H, _I), lambda n, meta: (meta[n], 0, 0)),
            pl.BlockSpec((1, _I, _H), lambda n, meta: (meta[n], 0, 0)),
            pl.BlockSpec((_BTR, _LN), lambda n, meta: (n, 0)),
        ],
        out_specs=pl.BlockSpec((_BTR, _H // 2), lambda n, meta: (n, 0)),
    ),
    out_shape=jax.ShapeDtypeStruct((_TPR, _H // 2), jnp.int32),
)

@functools.lru_cache(maxsize=None)
def _sc_kernels():
    mesh = plsc.VectorSubcoreMesh(core_axis_name="c", subcore_axis_name="s",
                                  num_cores=_NC, num_subcores=_NS)

    n_chunks = _TPW // _CH

    @functools.partial(
        pl.kernel,
        out_type=(
            jax.ShapeDtypeStruct((_TPR, _H // 2), jnp.int32),
            jax.ShapeDtypeStruct((_TPR, _LN), jnp.float32),
        ),
        mesh=mesh,
        scratch_types=[
            pltpu.VMEM((_TPW,), jnp.int32),
            pltpu.VMEM((_TPW,), jnp.int32),
            pltpu.VMEM((_TPW, _LN), jnp.float32),
            pltpu.VMEM((_TPW, _LN), jnp.float32),
            pltpu.VMEM((_CH, _H // 2), jnp.int32),
            pltpu.VMEM((_CH, _H // 2), jnp.int32),
            pltpu.SemaphoreType.DMA,
            pltpu.SemaphoreType.DMA,
            pltpu.SemaphoreType.DMA,
            pltpu.SemaphoreType.DMA,
            pltpu.SemaphoreType.DMA,
        ],
    )
    def sc_dispatch(xf_hbm, r0_hbm, r1_hbm, w0_hbm, w1_hbm, xs_hbm, ws_hbm,
                    r0_all, r1_all, wv0_all, wv1_all, xb0, xb1,
                    sem_in, semx0, semx1, sems0, sems1):
        wid = lax.axis_index("s") * _NC + lax.axis_index("c")
        base = wid * _TPW
        xbufs = (xb0, xb1)
        semx = (semx0, semx1)
        semsc = (sems0, sems1)
        pre = [
            pltpu.async_copy(r0_hbm.at[pl.ds(base, _TPW)], r0_all, sem_in),
            pltpu.async_copy(r1_hbm.at[pl.ds(base, _TPW)], r1_all, sem_in),
            pltpu.async_copy(w0_hbm.at[pl.ds(base, _TPW)], wv0_all, sem_in),
            pltpu.async_copy(w1_hbm.at[pl.ds(base, _TPW)], wv1_all, sem_in),
        ]
        xl = {0: pltpu.async_copy(xf_hbm.at[pl.ds(base, _CH)], xbufs[0],
                                  semx[0])}
        for cp in pre:
            cp.wait()
        sc_pend = {}
        for ci in range(n_chunks):
            b = ci % 2
            if ci >= 1:
                for cp in sc_pend.pop(ci - 1):
                    cp.wait()
            if ci + 1 < n_chunks:
                t1 = base + (ci + 1) * _CH
                xl[ci + 1] = pltpu.async_copy(
                    xf_hbm.at[pl.ds(t1, _CH)], xbufs[(ci + 1) % 2],
                    semx[(ci + 1) % 2])
            xl.pop(ci).wait()
            i0 = r0_all[pl.ds(ci * _CH, _CH)]
            i1 = r1_all[pl.ds(ci * _CH, _CH)]
            sc_pend[ci] = [
                pltpu.async_copy(xbufs[b], xs_hbm.at[i0], semsc[b]),
                pltpu.async_copy(xbufs[b], xs_hbm.at[i1], semsc[b]),
                pltpu.async_copy(wv0_all.at[pl.ds(ci * _CH, _CH)],
                                 ws_hbm.at[i0], semsc[b]),
                pltpu.async_copy(wv1_all.at[pl.ds(ci * _CH, _CH)],
                                 ws_hbm.at[i1], semsc[b]),
            ]
        for cp in sc_pend.pop(n_chunks - 1):
            cp.wait()

    n_jobs = 2 * (_TPW // _CH)

    @functools.partial(
        pl.kernel,
        out_type=(
            jax.ShapeDtypeStruct((_T, _H // 2), jnp.int32),
            jax.ShapeDtypeStruct((_T, _H // 2), jnp.int32),
        ),
        mesh=mesh,
        scratch_types=[
            pltpu.VMEM((_TPW,), jnp.int32),
            pltpu.VMEM((_TPW,), jnp.int32),
            pltpu.VMEM((_CH, _H // 2), jnp.int32),
            pltpu.VMEM((_CH, _H // 2), jnp.int32),
            pltpu.SemaphoreType.DMA,
            pltpu.SemaphoreType.DMA,
            pltpu.SemaphoreType.DMA,
            pltpu.SemaphoreType.DMA,
            pltpu.SemaphoreType.DMA,
        ],
    )
    def sc_combine(yr_hbm, r0_hbm, r1_hbm, g0_hbm, g1_hbm,
                   r0_all, r1_all, buf0, buf1, sem_in,
                   semg0, semg1, semo0, semo1):
        wid = lax.axis_index("s") * _NC + lax.axis_index("c")
        base = wid * _TPW
        bufs = (buf0, buf1)
        semg = (semg0, semg1)
        semo = (semo0, semo1)
        pre = [
            pltpu.async_copy(r0_hbm.at[pl.ds(base, _TPW)], r0_all, sem_in),
            pltpu.async_copy(r1_hbm.at[pl.ds(base, _TPW)], r1_all, sem_in),
        ]
        for cp in pre:
            cp.wait()

        def job_src(j):
            ci, k = divmod(j, 2)
            idx_ref = r0_all if k == 0 else r1_all
            dst = g0_hbm if k == 0 else g1_hbm
            return ci, idx_ref, dst

        def issue_gather(j):
            ci, idx_ref, _ = job_src(j)
            iv = idx_ref[pl.ds(ci * _CH, _CH)]
            return pltpu.async_copy(yr_hbm.at[iv], bufs[j % 2], semg[j % 2])

        g_pend = {0: issue_gather(0)}
        o_pend = {}
        for j in range(n_jobs):
            b = j % 2
            if j + 1 < n_jobs:
                if j >= 1:
                    o_pend.pop(j - 1).wait()
                g_pend[j + 1] = issue_gather(j + 1)
            g_pend.pop(j).wait()
            ci, _, dst = job_src(j)
            o_pend[j] = pltpu.async_copy(
                bufs[b], dst.at[pl.ds(base + ci * _CH, _CH)], semo[b])
        o_pend.pop(n_jobs - 1).wait()

    return sc_dispatch, sc_combine


def kernel(x, shared_gate, shared_up, shared_down, routed_gate, routed_up,
           routed_down, router_w, expert_bias):
    b, s, h = x.shape
    xf = x.reshape(-1, h)
    rw = jnp.pad(router_w, ((0, 0), (0, _LN - _E)))
    bias = jnp.pad(expert_bias, (0, _LN - _E)).reshape(1, _LN)
    r0, r1, w0, w1, meta, xb = _router_call(xf, rw, bias)
    r0f = r0.reshape(_T)
    r1f = r1.reshape(_T)
    meta_flat = meta.reshape(_LN)
    sc_dispatch, sc_combine = _sc_kernels()
    xs32, ws = sc_dispatch(xb, r0f, r1f, w0, w1)
    yr = _routed_call(meta_flat, xs32, routed_gate, routed_up,
                      routed_down, ws)
    g0_32, g1_32 = sc_combine(yr, r0f, r1f)
    out = _shared_call(xf, shared_gate, shared_up, shared_down,
                       g0_32, g1_32)
    aux_loss = jnp.asarray(0.0, dtype=x.dtype)
    return (out.reshape(b, s, h), aux_loss)


# speedup vs baseline: 4.3053x; 1.0032x over previous
import functools

import jax
import jax.numpy as jnp
from jax import lax
from jax.experimental import pallas as pl
from jax.experimental.pallas import tpu as pltpu
from jax.experimental.pallas import tpu_sc as plsc

_T, _H, _I, _E = 2048, 2048, 1024, 8
_BT = 256
_BTR = 256
_NTR = 24
_TPR = _NTR * _BTR
_LN = 128
_NC, _NS = 2, 16
_NW = _NC * _NS
_TPW = _T // _NW
_CH = 16


def _pack_bf16(a):
    n = a.shape[1] // 2
    lo = lax.bitcast_convert_type(a[:, :n].astype(jnp.bfloat16), jnp.uint16)
    hi = lax.bitcast_convert_type(a[:, n:].astype(jnp.bfloat16), jnp.uint16)
    w = lo.astype(jnp.uint32) | (hi.astype(jnp.uint32) << 16)
    return lax.bitcast_convert_type(w, jnp.int32)


def _unpack_bf16(w):
    wu = lax.bitcast_convert_type(w, jnp.uint32)
    lo = lax.bitcast_convert_type((wu & 0xFFFF).astype(jnp.uint16),
                                  jnp.bfloat16)
    hi = lax.bitcast_convert_type((wu >> 16).astype(jnp.uint16), jnp.bfloat16)
    return jnp.concatenate([lo.astype(jnp.float32), hi.astype(jnp.float32)],
                           axis=1)


def _router_body(x_ref, rw_ref, bias_ref, r0_ref, r1_ref, w0_ref, w1_ref,
                 meta_ref, xb_ref):
    x = x_ref[...]
    xb_ref[...] = _pack_bf16(x)
    logits = jnp.dot(x, rw_ref[...], preferred_element_type=jnp.float32)
    col = lax.broadcasted_iota(jnp.int32, (_T, _LN), 1)
    g = jax.nn.sigmoid(logits + bias_ref[...])
    g = jnp.where(col < _E, g, -1.0)
    top1 = jnp.max(g, axis=1, keepdims=True)
    i1 = jnp.min(jnp.where(g == top1, col, _LN), axis=1, keepdims=True)
    oh0 = col == i1
    g2 = jnp.where(oh0, -1.0, g)
    top2 = jnp.max(g2, axis=1, keepdims=True)
    i2 = jnp.min(jnp.where(g2 == top2, col, _LN), axis=1, keepdims=True)
    oh1 = col == i2
    den = top1 + top2 + 1e-9
    w0_ref[...] = jnp.broadcast_to(top1 / den, (_T, _LN))
    w1_ref[...] = jnp.broadcast_to(top2 / den, (_T, _LN))
    cnt = oh0.astype(jnp.float32) + oh1.astype(jnp.float32)
    counts = jnp.sum(cnt, axis=0, keepdims=True)
    padded = jnp.floor((counts + (_BTR - 1)) * (1.0 / _BTR)) * _BTR
    ea = lax.broadcasted_iota(jnp.int32, (_LN, _LN), 0)
    eb = lax.broadcasted_iota(jnp.int32, (_LN, _LN), 1)
    off = jnp.dot(padded, (ea < eb).astype(jnp.float32),
                  preferred_element_type=jnp.float32)
    ta = lax.broadcasted_iota(jnp.int32, (_T, _T), 0)
    tb = lax.broadcasted_iota(jnp.int32, (_T, _T), 1)
    cumex = jnp.dot((tb < ta).astype(jnp.float32), cnt,
                    preferred_element_type=jnp.float32)
    pos = off + cumex
    r0 = jnp.sum(jnp.where(oh0, pos, 0.0), axis=1, keepdims=True)
    r1 = jnp.sum(jnp.where(oh1, pos, 0.0), axis=1, keepdims=True)
    r0_ref[...] = r0.astype(jnp.int32)
    r1_ref[...] = r1.astype(jnp.int32)
    total = jnp.sum(padded)
    lane = lax.broadcasted_iota(jnp.int32, (1, _LN), 1)
    start = lane.astype(jnp.float32) * _BTR
    eid = jnp.zeros((1, _LN), jnp.int32)
    for e in range(_E):
        eid = eid + (start >= off[0, e] + padded[0, e]).astype(jnp.int32)
    eid = jnp.minimum(eid, _E - 1)
    valid = ((lane - 64).astype(jnp.float32) * _BTR < total).astype(jnp.int32)
    meta_ref[...] = jnp.where(lane < 64, eid, valid)


_router_call = pl.pallas_call(
    _router_body,
    out_shape=(
        jax.ShapeDtypeStruct((_T, 1), jnp.int32),
        jax.ShapeDtypeStruct((_T, 1), jnp.int32),
        jax.ShapeDtypeStruct((_T, _LN), jnp.float32),
        jax.ShapeDtypeStruct((_T, _LN), jnp.float32),
        jax.ShapeDtypeStruct((1, _LN), jnp.int32),
        jax.ShapeDtypeStruct((_T, _H // 2), jnp.int32),
    ),
)


def _shared_body(x_ref, gw_ref, uw_ref, dw_ref, g0_ref, g1_ref, y_ref):
    x = x_ref[...]
    g = jnp.dot(x, gw_ref[0], preferred_element_type=jnp.float32)
    u = jnp.dot(x, uw_ref[0], preferred_element_type=jnp.float32)
    a = (g * jax.nn.sigmoid(g)) * u
    y = jnp.dot(a, dw_ref[0], preferred_element_type=jnp.float32)
    y_ref[...] = y + _unpack_bf16(g0_ref[...]) + _unpack_bf16(g1_ref[...])


_shared_call = pl.pallas_call(
    _shared_body,
    grid=(_T // _BT,),
    in_specs=[
        pl.BlockSpec((_BT, _H), lambda n: (n, 0)),
        pl.BlockSpec((1, _H, _I), lambda n: (0, 0, 0)),
        pl.BlockSpec((1, _H, _I), lambda n: (0, 0, 0)),
        pl.BlockSpec((1, _I, _H), lambda n: (0, 0, 0)),
        pl.BlockSpec((_BT, _H // 2), lambda n: (n, 0)),
        pl.BlockSpec((_BT, _H // 2), lambda n: (n, 0)),
    ],
    out_specs=pl.BlockSpec((_BT, _H), lambda n: (n, 0)),
    out_shape=jax.ShapeDtypeStruct((_T, _H), jnp.float32),
)


def _routed_body(meta_ref, x_ref, gw_ref, uw_ref, dw_ref, w_ref, y_ref):
    n = pl.program_id(0)

    @pl.when(meta_ref[64 + n] == 1)
    def _():
        x = _unpack_bf16(x_ref[...])
        g = jnp.dot(x, gw_ref[0], preferred_element_type=jnp.float32)
        u = jnp.dot(x, uw_ref[0], preferred_element_type=jnp.float32)
        a = (g * jax.nn.sigmoid(g)) * u
        y = jnp.dot(a, dw_ref[0], preferred_element_type=jnp.float32)
        y_ref[...] = _pack_bf16(y * w_ref[:, 0:1])


_routed_call = pl.pallas_call(
    _routed_body,
    grid_spec=pltpu.PrefetchScalarGridSpec(
        num_scalar_prefetch=1,
        grid=(_NTR,),
        in_specs=[
            pl.BlockSpec((_BTR, _H // 2),
                         lambda n, meta: (jnp.where(meta[64 + n] == 1, n, 0),
                                          0)),
            pl.BlockSpec((1, _H, _I), lambda n, meta: (meta[n], 0, 0)),
            pl.BlockSpec((1, _H, _I), lambda n, meta: (meta[n], 0, 0)),
            pl.BlockSpec((1, _I, _H), lambda n, meta: (meta[n], 0, 0)),
            pl.BlockSpec((_BTR, _LN), lambda n, meta: (n, 0)),
        ],
        out_specs=pl.BlockSpec((_BTR, _H // 2), lambda n, meta: (n, 0)),
    ),
    out_shape=jax.ShapeDtypeStruct((_TPR, _H // 2), jnp.int32),
)

@functools.lru_cache(maxsize=None)
def _sc_kernels():
    mesh = plsc.VectorSubcoreMesh(core_axis_name="c", subcore_axis_name="s",
                                  num_cores=_NC, num_subcores=_NS)

    n_chunks = _TPW // _CH

    @functools.partial(
        pl.kernel,
        out_type=(
            jax.ShapeDtypeStruct((_TPR, _H // 2), jnp.int32),
            jax.ShapeDtypeStruct((_TPR, _LN), jnp.float32),
        ),
        mesh=mesh,
        scratch_types=[
            pltpu.VMEM((_TPW,), jnp.int32),
            pltpu.VMEM((_TPW,), jnp.int32),
            pltpu.VMEM((_TPW, _LN), jnp.float32),
            pltpu.VMEM((_TPW, _LN), jnp.float32),
            pltpu.VMEM((_CH, _H // 2), jnp.int32),
            pltpu.VMEM((_CH, _H // 2), jnp.int32),
            pltpu.SemaphoreType.DMA,
            pltpu.SemaphoreType.DMA,
            pltpu.SemaphoreType.DMA,
            pltpu.SemaphoreType.DMA,
            pltpu.SemaphoreType.DMA,
        ],
    )
    def sc_dispatch(xf_hbm, r0_hbm, r1_hbm, w0_hbm, w1_hbm, xs_hbm, ws_hbm,
                    r0_all, r1_all, wv0_all, wv1_all, xb0, xb1,
                    sem_in, semx0, semx1, sems0, sems1):
        wid = lax.axis_index("s") * _NC + lax.axis_index("c")
        base = wid * _TPW
        xbufs = (xb0, xb1)
        semx = (semx0, semx1)
        semsc = (sems0, sems1)
        pre = [
            pltpu.async_copy(r0_hbm.at[pl.ds(base, _TPW)], r0_all, sem_in),
            pltpu.async_copy(r1_hbm.at[pl.ds(base, _TPW)], r1_all, sem_in),
            pltpu.async_copy(w0_hbm.at[pl.ds(base, _TPW)], wv0_all, sem_in),
            pltpu.async_copy(w1_hbm.at[pl.ds(base, _TPW)], wv1_all, sem_in),
        ]
        xl = {0: pltpu.async_copy(xf_hbm.at[pl.ds(base, _CH)], xbufs[0],
                                  semx[0])}
        for cp in pre:
            cp.wait()
        sc_pend = {}
        for ci in range(n_chunks):
            b = ci % 2
            if ci >= 1:
                for cp in sc_pend.pop(ci - 1):
                    cp.wait()
            if ci + 1 < n_chunks:
                t1 = base + (ci + 1) * _CH
                xl[ci + 1] = pltpu.async_copy(
                    xf_hbm.at[pl.ds(t1, _CH)], xbufs[(ci + 1) % 2],
                    semx[(ci + 1) % 2])
            xl.pop(ci).wait()
            i0 = r0_all[pl.ds(ci * _CH, _CH)]
            i1 = r1_all[pl.ds(ci * _CH, _CH)]
            sc_pend[ci] = [
                pltpu.async_copy(xbufs[b], xs_hbm.at[i0], semsc[b]),
                pltpu.async_copy(xbufs[b], xs_hbm.at[i1], semsc[b]),
                pltpu.async_copy(wv0_all.at[pl.ds(ci * _CH, _CH)],
                                 ws_hbm.at[i0], semsc[b]),
                pltpu.async_copy(wv1_all.at[pl.ds(ci * _CH, _CH)],
                                 ws_hbm.at[i1], semsc[b]),
            ]
        for cp in sc_pend.pop(n_chunks - 1):
            cp.wait()

    n_jobs = 2 * (_TPW // _CH)

    @functools.partial(
        pl.kernel,
        out_type=(
            jax.ShapeDtypeStruct((_T, _H // 2), jnp.int32),
            jax.ShapeDtypeStruct((_T, _H // 2), jnp.int32),
        ),
        mesh=mesh,
        scratch_types=[
            pltpu.VMEM((_TPW,), jnp.int32),
            pltpu.VMEM((_TPW,), jnp.int32),
            pltpu.VMEM((_CH, _H // 2), jnp.int32),
            pltpu.VMEM((_CH, _H // 2), jnp.int32),
            pltpu.SemaphoreType.DMA,
            pltpu.SemaphoreType.DMA,
            pltpu.SemaphoreType.DMA,
            pltpu.SemaphoreType.DMA,
            pltpu.SemaphoreType.DMA,
        ],
    )
    def sc_combine(yr_hbm, r0_hbm, r1_hbm, g0_hbm, g1_hbm,
                   r0_all, r1_all, buf0, buf1, sem_in,
                   semg0, semg1, semo0, semo1):
        wid = lax.axis_index("s") * _NC + lax.axis_index("c")
        base = wid * _TPW
        bufs = (buf0, buf1)
        semg = (semg0, semg1)
        semo = (semo0, semo1)
        pre = [
            pltpu.async_copy(r0_hbm.at[pl.ds(base, _TPW)], r0_all, sem_in),
            pltpu.async_copy(r1_hbm.at[pl.ds(base, _TPW)], r1_all, sem_in),
        ]
        for cp in pre:
            cp.wait()

        def job_src(j):
            ci, k = divmod(j, 2)
            idx_ref = r0_all if k == 0 else r1_all
            dst = g0_hbm if k == 0 else g1_hbm
            return ci, idx_ref, dst

        def issue_gather(j):
            ci, idx_ref, _ = job_src(j)
            iv = idx_ref[pl.ds(ci * _CH, _CH)]
            return pltpu.async_copy(yr_hbm.at[iv], bufs[j % 2], semg[j % 2])

        g_pend = {0: issue_gather(0)}
        o_pend = {}
        for j in range(n_jobs):
            b = j % 2
            if j + 1 < n_jobs:
                if j >= 1:
                    o_pend.pop(j - 1).wait()
                g_pend[j + 1] = issue_gather(j + 1)
            g_pend.pop(j).wait()
            ci, _, dst = job_src(j)
            o_pend[j] = pltpu.async_copy(
                bufs[b], dst.at[pl.ds(base + ci * _CH, _CH)], semo[b])
        o_pend.pop(n_jobs - 1).wait()

    return sc_dispatch, sc_combine


def kernel(x, shared_gate, shared_up, shared_down, routed_gate, routed_up,
           routed_down, router_w, expert_bias):
    b, s, h = x.shape
    xf = x.reshape(-1, h)
    rw = jnp.pad(router_w, ((0, 0), (0, _LN - _E)))
    bias = jnp.pad(expert_bias, (0, _LN - _E)).reshape(1, _LN)
    r0, r1, w0, w1, meta, xb = _router_call(xf, rw, bias)
    r0f = r0.reshape(_T)
    r1f = r1.reshape(_T)
    meta_flat = meta.reshape(_LN)
    sc_dispatch, sc_combine = _sc_kernels()
    xs32, ws = sc_dispatch(xb, r0f, r1f, w0, w1)
    yr = _routed_call(meta_flat, xs32, routed_gate, routed_up,
                      routed_down, ws)
    g0_32, g1_32 = sc_combine(yr, r0f, r1f)
    out = _shared_call(xf, shared_gate, shared_up, shared_down,
                       g0_32, g1_32)
    aux_loss = jnp.asarray(0.0, dtype=x.dtype)
    return (out.reshape(b, s, h), aux_loss)
